# baseline (speedup 1.0000x reference)
import functools

import jax
import jax.numpy as jnp
from jax import lax
from jax.experimental import pallas as pl
from jax.experimental.pallas import tpu as pltpu
from jax.experimental.pallas import tpu_sc as plsc

BATCH = 16384
MF_DIM = 16
MLP_DIM = 32


def _sc_gather(uidx, midx, user_mf, movie_mf, user_mlp, movie_mlp):
    info = plsc.get_sparse_core_info()
    nc, ns = info.num_cores, info.num_subcores
    nw = nc * ns
    bpw = BATCH // nw
    lanes = info.num_lanes
    nvec = bpw // lanes
    mesh = plsc.VectorSubcoreMesh(core_axis_name="c", subcore_axis_name="s")

    @functools.partial(
        pl.kernel,
        mesh=mesh,
        out_type=[
            jax.ShapeDtypeStruct((BATCH, MF_DIM), jnp.float32),
            jax.ShapeDtypeStruct((BATCH, MF_DIM), jnp.float32),
            jax.ShapeDtypeStruct((BATCH, MLP_DIM), jnp.float32),
            jax.ShapeDtypeStruct((BATCH, MLP_DIM), jnp.float32),
        ],
        scratch_types=[
            pltpu.VMEM((bpw,), jnp.int32),
            pltpu.VMEM((bpw,), jnp.int32),
            pltpu.SemaphoreType.DMA,
        ],
    )
    def k(uidx_h, midx_h, umf_h, mmf_h, umlp_h, mmlp_h,
          umf_o, mmf_o, umlp_o, mmlp_o,
          uidx_v, midx_v, sem):
        wid = lax.axis_index("s") * nc + lax.axis_index("c")
        base = wid * bpw
        pltpu.sync_copy(uidx_h.at[pl.ds(base, bpw)], uidx_v)
        pltpu.sync_copy(midx_h.at[pl.ds(base, bpw)], midx_v)

        @pl.loop(0, nvec)
        def chunk(v):
            uvec = uidx_v[pl.ds(v * lanes, lanes)]
            mvec = midx_v[pl.ds(v * lanes, lanes)]
            copies = []
            for lane in range(lanes):
                u = uvec[lane]
                m = mvec[lane]
                i = base + v * lanes + lane
                copies.append(pltpu.async_copy(umf_h.at[u], umf_o.at[i], sem))
                copies.append(pltpu.async_copy(mmf_h.at[m], mmf_o.at[i], sem))
                copies.append(pltpu.async_copy(umlp_h.at[u], umlp_o.at[i], sem))
                copies.append(pltpu.async_copy(mmlp_h.at[m], mmlp_o.at[i], sem))
            for c in copies:
                c.wait()

    return k(uidx, midx, user_mf, movie_mf, user_mlp, movie_mlp)


def _mlp_body(umf_ref, mmf_ref, umlp_ref, mmlp_ref,
              w1u_ref, w1m_ref, b1_ref, w2_ref, b2_ref, w3_ref, b3_ref,
              wfa_ref, wfb_ref, bf_ref, out_ref):
    h1 = jnp.maximum(
        jnp.dot(umlp_ref[...], w1u_ref[...], preferred_element_type=jnp.float32)
        + jnp.dot(mmlp_ref[...], w1m_ref[...], preferred_element_type=jnp.float32)
        + b1_ref[...], 0.0)
    h2 = jnp.maximum(
        jnp.dot(h1, w2_ref[...], preferred_element_type=jnp.float32)
        + b2_ref[...], 0.0)
    h3 = jnp.maximum(
        jnp.dot(h2, w3_ref[...], preferred_element_type=jnp.float32)
        + b3_ref[...], 0.0)
    gmf = umf_ref[...] * mmf_ref[...]
    fin = (jnp.dot(gmf, wfa_ref[...], preferred_element_type=jnp.float32)
           + jnp.dot(h3, wfb_ref[...], preferred_element_type=jnp.float32)
           + bf_ref[0, 0])
    out_ref[...] = 5.0 * jax.nn.sigmoid(fin)


def _tc_mlp(umf, mmf, umlp, mmlp, w1u, w1m, b1, w2t, b2, w3t, b3, wfa, wfb, bf):
    bk = 4096
    grid = (BATCH // bk,)
    full = lambda i: (0, 0)
    row = lambda i: (i, 0)
    return pl.pallas_call(
        _mlp_body,
        grid=grid,
        in_specs=[
            pl.BlockSpec((bk, MF_DIM), row),
            pl.BlockSpec((bk, MF_DIM), row),
            pl.BlockSpec((bk, MLP_DIM), row),
            pl.BlockSpec((bk, MLP_DIM), row),
            pl.BlockSpec((MLP_DIM, 2 * MLP_DIM), full),
            pl.BlockSpec((MLP_DIM, 2 * MLP_DIM), full),
            pl.BlockSpec((1, 2 * MLP_DIM), full),
            pl.BlockSpec((2 * MLP_DIM, 2 * MLP_DIM), full),
            pl.BlockSpec((1, 2 * MLP_DIM), full),
            pl.BlockSpec((2 * MLP_DIM, MLP_DIM), full),
            pl.BlockSpec((1, MLP_DIM), full),
            pl.BlockSpec((MF_DIM, 1), full),
            pl.BlockSpec((MLP_DIM, 1), full),
            pl.BlockSpec((1, 1), full),
        ],
        out_specs=pl.BlockSpec((bk, 1), row),
        out_shape=jax.ShapeDtypeStruct((BATCH, 1), jnp.float32),
    )(umf, mmf, umlp, mmlp, w1u, w1m, b1, w2t, b2, w3t, b3, wfa, wfb, bf)


def kernel(X, user_mf, movie_mf, user_mlp, movie_mlp,
           W1, b1, W2, b2, W3, b3, Wf, bf):
    uidx = X[:, 0]
    midx = X[:, 1]
    umf, mmf, umlp, mmlp = _sc_gather(uidx, midx, user_mf, movie_mf,
                                      user_mlp, movie_mlp)
    w1t = W1.T
    w1u = w1t[:MLP_DIM, :]
    w1m = w1t[MLP_DIM:, :]
    wft = Wf.T
    wfa = wft[:MF_DIM, :]
    wfb = wft[MF_DIM:, :]
    return _tc_mlp(umf, mmf, umlp, mmlp,
                   w1u, w1m, b1.reshape(1, -1), W2.T, b2.reshape(1, -1),
                   W3.T, b3.reshape(1, -1), wfa, wfb, bf.reshape(1, 1))


# speedup vs baseline: 16.3946x; 16.3946x over previous
import functools

import jax
import jax.numpy as jnp
from jax import lax
from jax.experimental import pallas as pl
from jax.experimental.pallas import tpu as pltpu
from jax.experimental.pallas import tpu_sc as plsc

BATCH = 16384
MF_DIM = 16
MLP_DIM = 32


def _sc_gather(uidx, midx, user_mf, movie_mf, user_mlp, movie_mlp):
    info = plsc.get_sparse_core_info()
    nc, ns = info.num_cores, info.num_subcores
    nw = nc * ns
    bpw = BATCH // nw
    lanes = info.num_lanes
    nvec = bpw // lanes
    mesh = plsc.VectorSubcoreMesh(core_axis_name="c", subcore_axis_name="s")

    @functools.partial(
        pl.kernel,
        mesh=mesh,
        out_type=[
            jax.ShapeDtypeStruct((BATCH, MF_DIM), jnp.float32),
            jax.ShapeDtypeStruct((BATCH, MF_DIM), jnp.float32),
            jax.ShapeDtypeStruct((BATCH, MLP_DIM), jnp.float32),
            jax.ShapeDtypeStruct((BATCH, MLP_DIM), jnp.float32),
        ],
        scratch_types=[
            pltpu.VMEM((bpw,), jnp.int32),
            pltpu.VMEM((bpw,), jnp.int32),
            pltpu.SemaphoreType.DMA,
        ],
    )
    def k(uidx_h, midx_h, umf_h, mmf_h, umlp_h, mmlp_h,
          umf_o, mmf_o, umlp_o, mmlp_o,
          uidx_v, midx_v, sem):
        wid = lax.axis_index("s") * nc + lax.axis_index("c")
        base = wid * bpw
        pltpu.sync_copy(uidx_h.at[pl.ds(base, bpw)], uidx_v)
        pltpu.sync_copy(midx_h.at[pl.ds(base, bpw)], midx_v)

        @pl.loop(0, nvec)
        def chunk(v):
            uvec = uidx_v[pl.ds(v * lanes, lanes)]
            mvec = midx_v[pl.ds(v * lanes, lanes)]
            copies = []
            for lane in range(lanes):
                u = uvec[lane]
                m = mvec[lane]
                i = base + v * lanes + lane
                copies.append(pltpu.async_copy(umf_h.at[u], umf_o.at[i], sem))
                copies.append(pltpu.async_copy(mmf_h.at[m], mmf_o.at[i], sem))
                copies.append(pltpu.async_copy(umlp_h.at[u], umlp_o.at[i], sem))
                copies.append(pltpu.async_copy(mmlp_h.at[m], mmlp_o.at[i], sem))
            for c in copies:
                c.wait()

    return k(uidx, midx, user_mf, movie_mf, user_mlp, movie_mlp)


def _mlp_body(umf_ref, mmf_ref, umlp_ref, mmlp_ref,
              w1u_ref, w1m_ref, b1_ref, w2_ref, b2_ref, w3_ref, b3_ref,
              wfa_ref, wfb_ref, bf_ref, out_ref):
    h1 = jnp.maximum(
        jnp.dot(umlp_ref[...], w1u_ref[...], preferred_element_type=jnp.float32)
        + jnp.dot(mmlp_ref[...], w1m_ref[...], preferred_element_type=jnp.float32)
        + b1_ref[...], 0.0)
    h2 = jnp.maximum(
        jnp.dot(h1, w2_ref[...], preferred_element_type=jnp.float32)
        + b2_ref[...], 0.0)
    h3 = jnp.maximum(
        jnp.dot(h2, w3_ref[...], preferred_element_type=jnp.float32)
        + b3_ref[...], 0.0)
    gmf = umf_ref[...] * mmf_ref[...]
    fin = (jnp.dot(gmf, wfa_ref[...], preferred_element_type=jnp.float32)
           + jnp.dot(h3, wfb_ref[...], preferred_element_type=jnp.float32)
           + bf_ref[0, 0])
    out_ref[...] = 5.0 * jax.nn.sigmoid(fin)


def _tc_mlp(umf, mmf, umlp, mmlp, w1u, w1m, b1, w2t, b2, w3t, b3, wfa, wfb, bf):
    bk = 4096
    grid = (BATCH // bk,)
    full = lambda i: (0, 0)
    row = lambda i: (i, 0)
    return pl.pallas_call(
        _mlp_body,
        grid=grid,
        in_specs=[
            pl.BlockSpec((bk, MF_DIM), row),
            pl.BlockSpec((bk, MF_DIM), row),
            pl.BlockSpec((bk, MLP_DIM), row),
            pl.BlockSpec((bk, MLP_DIM), row),
            pl.BlockSpec((MLP_DIM, 2 * MLP_DIM), full),
            pl.BlockSpec((MLP_DIM, 2 * MLP_DIM), full),
            pl.BlockSpec((1, 2 * MLP_DIM), full),
            pl.BlockSpec((2 * MLP_DIM, 2 * MLP_DIM), full),
            pl.BlockSpec((1, 2 * MLP_DIM), full),
            pl.BlockSpec((2 * MLP_DIM, MLP_DIM), full),
            pl.BlockSpec((1, MLP_DIM), full),
            pl.BlockSpec((MF_DIM, 1), full),
            pl.BlockSpec((MLP_DIM, 1), full),
            pl.BlockSpec((1, 1), full),
        ],
        out_specs=pl.BlockSpec((bk, 1), row),
        out_shape=jax.ShapeDtypeStruct((BATCH, 1), jnp.float32),
    )(umf, mmf, umlp, mmlp, w1u, w1m, b1, w2t, b2, w3t, b3, wfa, wfb, bf)


def kernel(X, user_mf, movie_mf, user_mlp, movie_mlp,
           W1, b1, W2, b2, W3, b3, Wf, bf):
    uidx = X[:, 0]
    midx = X[:, 1]
    umf = jnp.take(user_mf, uidx, axis=0, mode="clip")
    mmf = jnp.take(movie_mf, midx, axis=0, mode="clip")
    umlp = jnp.take(user_mlp, uidx, axis=0, mode="clip")
    mmlp = jnp.take(movie_mlp, midx, axis=0, mode="clip")
    w1t = W1.T
    w1u = w1t[:MLP_DIM, :]
    w1m = w1t[MLP_DIM:, :]
    wft = Wf.T
    wfa = wft[:MF_DIM, :]
    wfb = wft[MF_DIM:, :]
    return _tc_mlp(umf, mmf, umlp, mmlp,
                   w1u, w1m, b1.reshape(1, -1), W2.T, b2.reshape(1, -1),
                   W3.T, b3.reshape(1, -1), wfa, wfb, bf.reshape(1, 1))


# speedup vs baseline: 20.3904x; 1.2437x over previous
import jax
import jax.numpy as jnp
from jax.experimental import pallas as pl

BATCH = 16384
MF_DIM = 16
MLP_DIM = 32


def _mlp_body(umfT_ref, mmfT_ref, umlpT_ref, mmlpT_ref,
              w1a_ref, w1b_ref, b1_ref, w2_ref, b2_ref, w3_ref, b3_ref,
              wfa_ref, wfb_ref, bf_ref, out_ref):
    h1 = jnp.maximum(
        jnp.dot(w1a_ref[...], umlpT_ref[...], preferred_element_type=jnp.float32)
        + jnp.dot(w1b_ref[...], mmlpT_ref[...], preferred_element_type=jnp.float32)
        + b1_ref[...], 0.0)
    h2 = jnp.maximum(
        jnp.dot(w2_ref[...], h1, preferred_element_type=jnp.float32)
        + b2_ref[...], 0.0)
    h3 = jnp.maximum(
        jnp.dot(w3_ref[...], h2, preferred_element_type=jnp.float32)
        + b3_ref[...], 0.0)
    gmf = umfT_ref[...] * mmfT_ref[...]
    fin = (jnp.dot(wfa_ref[...], gmf, preferred_element_type=jnp.float32)
           + jnp.dot(wfb_ref[...], h3, preferred_element_type=jnp.float32)
           + bf_ref[0, 0])
    out_ref[...] = 5.0 * jax.nn.sigmoid(fin)


def _tc_mlp(umfT, mmfT, umlpT, mmlpT, w1a, w1b, b1, w2, b2, w3, b3,
            wfa, wfb, bf):
    bk = 2048
    grid = (BATCH // bk,)
    full = lambda i: (0, 0)
    col = lambda i: (0, i)
    return pl.pallas_call(
        _mlp_body,
        grid=grid,
        in_specs=[
            pl.BlockSpec((MF_DIM, bk), col),
            pl.BlockSpec((MF_DIM, bk), col),
            pl.BlockSpec((MLP_DIM, bk), col),
            pl.BlockSpec((MLP_DIM, bk), col),
            pl.BlockSpec((2 * MLP_DIM, MLP_DIM), full),
            pl.BlockSpec((2 * MLP_DIM, MLP_DIM), full),
            pl.BlockSpec((2 * MLP_DIM, 1), full),
            pl.BlockSpec((2 * MLP_DIM, 2 * MLP_DIM), full),
            pl.BlockSpec((2 * MLP_DIM, 1), full),
            pl.BlockSpec((MLP_DIM, 2 * MLP_DIM), full),
            pl.BlockSpec((MLP_DIM, 1), full),
            pl.BlockSpec((1, MF_DIM), full),
            pl.BlockSpec((1, MLP_DIM), full),
            pl.BlockSpec((1, 1), full),
        ],
        out_specs=pl.BlockSpec((1, bk), col),
        out_shape=jax.ShapeDtypeStruct((1, BATCH), jnp.float32),
    )(umfT, mmfT, umlpT, mmlpT, w1a, w1b, b1, w2, b2, w3, b3, wfa, wfb, bf)


def kernel(X, user_mf, movie_mf, user_mlp, movie_mlp,
           W1, b1, W2, b2, W3, b3, Wf, bf):
    uidx = X[:, 0]
    midx = X[:, 1]
    umfT = jnp.take(user_mf, uidx, axis=0, mode="clip").T
    mmfT = jnp.take(movie_mf, midx, axis=0, mode="clip").T
    umlpT = jnp.take(user_mlp, uidx, axis=0, mode="clip").T
    mmlpT = jnp.take(movie_mlp, midx, axis=0, mode="clip").T
    w1a = W1[:, :MLP_DIM]
    w1b = W1[:, MLP_DIM:]
    wfa = Wf[:, :MF_DIM]
    wfb = Wf[:, MF_DIM:]
    out = _tc_mlp(umfT, mmfT, umlpT, mmlpT,
                  w1a, w1b, b1.reshape(-1, 1), W2, b2.reshape(-1, 1),
                  W3, b3.reshape(-1, 1), wfa, wfb, bf.reshape(1, 1))
    return out.reshape(BATCH, 1)
